# baseline (speedup 1.0000x reference)
import jax
import jax.numpy as jnp
from jax.experimental import pallas as pl
from jax.experimental.pallas import tpu as pltpu

_EPSILON = 0.05
_REG = 0.1
_NUM_ITER = 10
_B = 4096
_D = 1024
_TC = 1024
_NTC = _B // _TC
_R = 512
_NC = _B // _R


def _bf16_sum0(x):
    rows = x.shape[0]
    while rows > 16:
        rows //= 2
        x = x[:rows, :] + x[rows:2 * rows, :]
    return jnp.sum(x.astype(jnp.float32), axis=0, keepdims=True)


def _bf16_sum1(x):
    cols = x.shape[1]
    while cols > 256:
        cols //= 2
        x = x[:, :cols] + x[:, cols:2 * cols]
    return jnp.sum(x.astype(jnp.float32), axis=1, keepdims=True)


def _wloss_kernel(a_hbm, t_hbm, out_ref, k_ref, tT_ref, stage_ref, rt_ref,
                  u_ref, v_ref, tsems, asems):
    fB = jnp.float32(1.0 / _B)

    for c in range(_NTC):
        pltpu.make_async_copy(t_hbm.at[pl.ds(c * _TC, _TC), :],
                              stage_ref.at[pl.ds(c * _TC, _TC), :],
                              tsems.at[c]).start()
    for c in range(_NTC):
        pltpu.make_async_copy(t_hbm.at[pl.ds(c * _TC, _TC), :],
                              stage_ref.at[pl.ds(c * _TC, _TC), :],
                              tsems.at[c]).wait()
        xb = stage_ref[c * _TC:(c + 1) * _TC, :].astype(jnp.bfloat16)
        xt = jnp.swapaxes(xb, 0, 1)
        tT_ref[:, c * _TC:(c + 1) * _TC] = xt
        xtf = xt.astype(jnp.float32)
        rt_ref[:, c * _TC:(c + 1) * _TC] = jax.lax.rsqrt(
            jnp.sum(xtf * xtf, axis=0, keepdims=True))
        pltpu.make_async_copy(a_hbm.at[pl.ds(c * _TC, _TC), :],
                              stage_ref.at[pl.ds(c * _TC, _TC), :],
                              asems.at[c]).start()
    rt = rt_ref[...]

    def mm_body(c, gmin):
        pltpu.make_async_copy(a_hbm.at[pl.ds(c * _TC, _TC), :],
                              stage_ref.at[pl.ds(c * _TC, _TC), :],
                              asems.at[c]).wait()
        af = stage_ref[pl.ds(c * _TC, _TC), :]
        ra = jax.lax.rsqrt(jnp.sum(af * af, axis=1, keepdims=True))
        g = jax.lax.dot_general(
            af.astype(jnp.bfloat16), tT_ref[...],
            dimension_numbers=(((1,), (0,)), ((), ())),
            preferred_element_type=jnp.float32)
        g = g * ra * rt
        k_ref[pl.ds(c * _TC, _TC), :] = g.astype(jnp.bfloat16)
        return jnp.minimum(gmin, jnp.min(g))

    gmin = jax.lax.fori_loop(0, _NTC, mm_body, jnp.float32(2.0))

    cexp2 = 1.4426950408889634 / (_EPSILON * (1.0 - gmin))

    cexp2_bf = cexp2.astype(jnp.bfloat16)

    def exp_body(c, acc):
        g = k_ref[pl.ds(c * _R, _R), :]
        e = jnp.exp2(g * cexp2_bf)
        k_ref[pl.ds(c * _R, _R), :] = e
        return acc + _bf16_sum0(e)

    csum = jax.lax.fori_loop(0, _NC, exp_body,
                             jnp.zeros((1, _B), jnp.float32))

    def bf_col_pass(c, acc):
        k = k_ref[pl.ds(c * _R, _R), :]
        uc = u_ref[pl.ds(c * _R, _R), :]
        return acc + _bf16_sum0(k * uc)

    def rowcol_scan(c, acc):
        k = k_ref[pl.ds(c * _R, _R), :]
        kv = _bf16_sum1(k * v_ref[...])
        u = (fB / kv).astype(jnp.bfloat16)
        return acc + _bf16_sum0(k * u)

    v_ref[...] = (1.0 / csum).astype(jnp.bfloat16)

    def sink_body(_it, carry):
        ktu = jax.lax.fori_loop(0, _NC, rowcol_scan,
                                jnp.zeros((1, _B), jnp.float32))
        v_ref[...] = (fB / ktu).astype(jnp.bfloat16)
        return carry

    jax.lax.fori_loop(0, _NUM_ITER - 1, sink_body, 0)

    def ce_pass(c, accs):
        ce_acc, klr_acc = accs
        k = k_ref[pl.ds(c * _R, _R), :]
        kv = _bf16_sum1(k * v_ref[...])
        u = fB / kv
        u_ref[pl.ds(c * _R, _R), :] = u.astype(jnp.bfloat16)
        row = u * kv
        klr = jnp.where(row > 0, row * (jnp.log(row) - fB), 0.0)
        klr_acc = klr_acc + jnp.sum(klr)
        off = pl.multiple_of(c * _R, _R)
        kd = k_ref[pl.ds(off, _R), pl.ds(off, _R)]
        vd = v_ref[:, pl.ds(off, _R)]
        tb = (kd * vd).astype(jnp.float32)
        eye = (jax.lax.broadcasted_iota(jnp.int32, (_R, _R), 0)
               == jax.lax.broadcasted_iota(jnp.int32, (_R, _R), 1))
        d = jnp.sum(jnp.where(eye, tb, 0.0), axis=1, keepdims=True)
        lse = jnp.log(jnp.float32(_B) + row)
        ce_acc = ce_acc + jnp.sum(lse - u * d)
        return (ce_acc, klr_acc)

    ce_sum, klr_sum = jax.lax.fori_loop(
        0, _NC, ce_pass, (jnp.float32(0.0), jnp.float32(0.0)))
    ce = ce_sum * fB
    kl_row = klr_sum * fB

    ktu_f = jax.lax.fori_loop(0, _NC, bf_col_pass,
                              jnp.zeros((1, _B), jnp.float32))
    col = v_ref[...].astype(jnp.float32) * ktu_f
    klc = jnp.where(col > 0, col * (jnp.log(col) - fB), 0.0)
    kl_col = jnp.sum(klc) * fB

    out_ref[0, 0] = ce + _REG * (kl_col + kl_row)


def kernel(audio_emb, text_emb, labels):
    del labels
    out = pl.pallas_call(
        _wloss_kernel,
        out_shape=jax.ShapeDtypeStruct((1, 1), jnp.float32),
        in_specs=[pl.BlockSpec(memory_space=pl.ANY),
                  pl.BlockSpec(memory_space=pl.ANY)],
        out_specs=pl.BlockSpec(memory_space=pltpu.SMEM),
        scratch_shapes=[
            pltpu.VMEM((_B, _B), jnp.bfloat16),
            pltpu.VMEM((_D, _B), jnp.bfloat16),
            pltpu.VMEM((_B, _D), jnp.float32),
            pltpu.VMEM((1, _B), jnp.float32),
            pltpu.VMEM((_B, 1), jnp.bfloat16),
            pltpu.VMEM((1, _B), jnp.bfloat16),
            pltpu.SemaphoreType.DMA((_NTC,)),
            pltpu.SemaphoreType.DMA((_NTC,)),
        ],
        compiler_params=pltpu.CompilerParams(
            vmem_limit_bytes=100 * 1024 * 1024),
    )(audio_emb, text_emb)
    return out[0, 0]


# speedup vs baseline: 1.0975x; 1.0975x over previous
import jax
import jax.numpy as jnp
from jax.experimental import pallas as pl
from jax.experimental.pallas import tpu as pltpu

_EPSILON = 0.05
_REG = 0.1
_NUM_ITER = 10
_B = 4096
_D = 1024
_TC = 1024
_NTC = _B // _TC
_R = 1024
_NC = _B // _R


def _bf16_sum0(x):
    rows = x.shape[0]
    while rows > 16:
        rows //= 2
        x = x[:rows, :] + x[rows:2 * rows, :]
    return jnp.sum(x.astype(jnp.float32), axis=0, keepdims=True)


def _bf16_sum1(x):
    cols = x.shape[1]
    while cols > 256:
        cols //= 2
        x = x[:, :cols] + x[:, cols:2 * cols]
    return jnp.sum(x.astype(jnp.float32), axis=1, keepdims=True)


def _wloss_kernel(a_hbm, t_hbm, out_ref, k_ref, tT_ref, stage_ref, rt_ref,
                  v_ref, tsems, asems):
    fB = jnp.float32(1.0 / _B)

    for c in range(_NTC):
        pltpu.make_async_copy(t_hbm.at[pl.ds(c * _TC, _TC), :],
                              stage_ref.at[pl.ds(c * _TC, _TC), :],
                              tsems.at[c]).start()
    for c in range(_NTC):
        pltpu.make_async_copy(t_hbm.at[pl.ds(c * _TC, _TC), :],
                              stage_ref.at[pl.ds(c * _TC, _TC), :],
                              tsems.at[c]).wait()
        xb = stage_ref[c * _TC:(c + 1) * _TC, :].astype(jnp.bfloat16)
        xt = jnp.swapaxes(xb, 0, 1)
        tT_ref[:, c * _TC:(c + 1) * _TC] = xt
        xtf = xt.astype(jnp.float32)
        rt_ref[:, c * _TC:(c + 1) * _TC] = jax.lax.rsqrt(
            jnp.sum(xtf * xtf, axis=0, keepdims=True))
        pltpu.make_async_copy(a_hbm.at[pl.ds(c * _TC, _TC), :],
                              stage_ref.at[pl.ds(c * _TC, _TC), :],
                              asems.at[c]).start()
    rt = rt_ref[...]

    def mm_body(c, gmin):
        pltpu.make_async_copy(a_hbm.at[pl.ds(c * _TC, _TC), :],
                              stage_ref.at[pl.ds(c * _TC, _TC), :],
                              asems.at[c]).wait()
        af = stage_ref[pl.ds(c * _TC, _TC), :]
        ra = jax.lax.rsqrt(jnp.sum(af * af, axis=1, keepdims=True))
        g = jax.lax.dot_general(
            af.astype(jnp.bfloat16), tT_ref[...],
            dimension_numbers=(((1,), (0,)), ((), ())),
            preferred_element_type=jnp.float32)
        g = g * ra * rt
        gb = g.astype(jnp.bfloat16)
        k_ref[pl.ds(c * _TC, _TC), :] = gb
        return jnp.minimum(gmin, jnp.min(gb, axis=0, keepdims=True))

    gmin_v = jax.lax.fori_loop(0, _NTC, mm_body,
                               jnp.full((1, _B), 2.0, jnp.bfloat16))
    gmin = jnp.min(gmin_v.astype(jnp.float32))

    cexp2 = 1.4426950408889634 / (_EPSILON * (1.0 - gmin))

    cexp2_bf = cexp2.astype(jnp.bfloat16)

    def exp_body(c, acc):
        g = k_ref[pl.ds(c * _R, _R), :]
        e = jnp.exp2(g * cexp2_bf)
        k_ref[pl.ds(c * _R, _R), :] = e
        return acc + _bf16_sum0(e)

    csum = jax.lax.fori_loop(0, _NC, exp_body,
                             jnp.zeros((1, _B), jnp.float32))

    def rowcol_scan(c, acc):
        k = k_ref[pl.ds(c * _R, _R), :]
        kv = _bf16_sum1(k * v_ref[...])
        u = (fB / kv).astype(jnp.bfloat16)
        return acc + _bf16_sum0(k * u)

    v_ref[...] = (1.0 / csum).astype(jnp.bfloat16)

    def sink_body(_it, carry):
        ktu = jax.lax.fori_loop(0, _NC, rowcol_scan,
                                jnp.zeros((1, _B), jnp.float32))
        v_ref[...] = (fB / ktu).astype(jnp.bfloat16)
        return carry

    jax.lax.fori_loop(0, _NUM_ITER - 1, sink_body, 0)

    def ce_pass(c, accs):
        ce_acc, klr_acc, ktu_acc = accs
        k = k_ref[pl.ds(c * _R, _R), :]
        kv = _bf16_sum1(k * v_ref[...])
        u = fB / kv
        ub = u.astype(jnp.bfloat16)
        ktu_acc = ktu_acc + _bf16_sum0(k * ub)
        row = u * kv
        klr = jnp.where(row > 0, row * (jnp.log(row) - fB), 0.0)
        klr_acc = klr_acc + jnp.sum(klr)
        off = pl.multiple_of(c * _R, _R)
        kd = k_ref[pl.ds(off, _R), pl.ds(off, _R)]
        vd = v_ref[:, pl.ds(off, _R)]
        tb = (kd * vd).astype(jnp.float32)
        eye = (jax.lax.broadcasted_iota(jnp.int32, (_R, _R), 0)
               == jax.lax.broadcasted_iota(jnp.int32, (_R, _R), 1))
        d = jnp.sum(jnp.where(eye, tb, 0.0), axis=1, keepdims=True)
        lse = jnp.log(jnp.float32(_B) + row)
        ce_acc = ce_acc + jnp.sum(lse - u * d)
        return (ce_acc, klr_acc, ktu_acc)

    ce_sum, klr_sum, ktu_f = jax.lax.fori_loop(
        0, _NC, ce_pass, (jnp.float32(0.0), jnp.float32(0.0),
                          jnp.zeros((1, _B), jnp.float32)))
    ce = ce_sum * fB
    kl_row = klr_sum * fB

    col = v_ref[...].astype(jnp.float32) * ktu_f
    klc = jnp.where(col > 0, col * (jnp.log(col) - fB), 0.0)
    kl_col = jnp.sum(klc) * fB

    out_ref[0, 0] = ce + _REG * (kl_col + kl_row)


def kernel(audio_emb, text_emb, labels):
    del labels
    out = pl.pallas_call(
        _wloss_kernel,
        out_shape=jax.ShapeDtypeStruct((1, 1), jnp.float32),
        in_specs=[pl.BlockSpec(memory_space=pl.ANY),
                  pl.BlockSpec(memory_space=pl.ANY)],
        out_specs=pl.BlockSpec(memory_space=pltpu.SMEM),
        scratch_shapes=[
            pltpu.VMEM((_B, _B), jnp.bfloat16),
            pltpu.VMEM((_D, _B), jnp.bfloat16),
            pltpu.VMEM((_B, _D), jnp.float32),
            pltpu.VMEM((1, _B), jnp.float32),
            pltpu.VMEM((1, _B), jnp.bfloat16),
            pltpu.SemaphoreType.DMA((_NTC,)),
            pltpu.SemaphoreType.DMA((_NTC,)),
        ],
        compiler_params=pltpu.CompilerParams(
            vmem_limit_bytes=100 * 1024 * 1024),
    )(audio_emb, text_emb)
    return out[0, 0]
